# speedup vs baseline: 99.1406x; 99.1406x over previous
import functools

import jax
import jax.numpy as jnp
from jax import lax
from jax.experimental import pallas as pl
from jax.experimental.pallas import tpu as pltpu
from jax.experimental.pallas import tpu_sc as plsc

_TAU = 5.0
_LAMBDA_MSE = 0.5

_S = 16384
_K = 200
_FLAT = _S * _K
_IDXW = 128
_ROWS = _FLAT // _IDXW
_NW = 32
_ROWS_PER_W = _ROWS // _NW
_G = 16
_STEPS = _ROWS_PER_W // _G


def _sc_gather(idx2d, theta):
    mesh = plsc.VectorSubcoreMesh(core_axis_name="c", subcore_axis_name="s")

    @functools.partial(
        pl.kernel,
        mesh=mesh,
        out_type=jax.ShapeDtypeStruct((_ROWS, _IDXW), jnp.float32),
        scratch_types=[
            pltpu.VMEM((_G, _IDXW), jnp.int32),
            pltpu.VMEM((_G, _IDXW), jnp.float32),
            pltpu.SemaphoreType.DMA,
        ],
    )
    def gather_kernel(idx_hbm, theta_hbm, out_hbm, idx_v, val_v, sem):
        nc = lax.axis_size("c")
        wid = lax.axis_index("s") * nc + lax.axis_index("c")
        base = wid * _ROWS_PER_W

        def step(g, carry):
            r0 = base + g * _G
            pltpu.sync_copy(idx_hbm.at[pl.ds(r0, _G)], idx_v)
            descs = []
            for j in range(_G):
                descs.append(
                    pltpu.async_copy(theta_hbm.at[idx_v.at[j]], val_v.at[j], sem)
                )
            for d in descs:
                d.wait()
            pltpu.sync_copy(val_v, out_hbm.at[pl.ds(r0, _G)])
            return carry

        lax.fori_loop(0, _STEPS, step, 0)

    return gather_kernel(idx2d, theta)


def _tc_body(a_ref, lens_ref, b_ref, t_ref, sc_ref, out_ref, acc_ref):
    i = pl.program_id(0)
    n = pl.num_programs(0)

    @pl.when(i == 0)
    def _init():
        acc_ref[0] = 0.0
        acc_ref[1] = 0.0
        acc_ref[2] = 0.0
        acc_ref[3] = 0.0

    t = t_ref[...] * _TAU
    bs = t.shape[0]
    kio = lax.broadcasted_iota(jnp.int32, (bs, _K), 1)
    mask = kio < lens_ref[...]
    maskf = mask.astype(jnp.float32)

    e = jnp.where(mask, jnp.exp(t), 0.0)
    rj = lax.broadcasted_iota(jnp.int32, (_K, _K), 0)
    ci = lax.broadcasted_iota(jnp.int32, (_K, _K), 1)
    tri = (rj >= ci).astype(jnp.float32)
    cumexp = jnp.dot(e, tri, preferred_element_type=jnp.float32)
    logc = jnp.log(cumexp + 1e-12)

    sum_t = jnp.sum(t * maskf)
    sum_lc = jnp.sum(logc * maskf)

    sc = sc_ref[...]
    w = jnp.maximum(1.0 / (1.0 + jnp.exp(-(sc - 0.5))), 0.1)
    wm = w * maskf
    pred = a_ref[0, 0] * t + b_ref[...]
    d = pred - sc * _TAU
    sum_se = jnp.sum(d * d * wm)
    sum_wm = jnp.sum(wm)

    acc_ref[0] += sum_t
    acc_ref[1] += sum_lc
    acc_ref[2] += sum_se
    acc_ref[3] += sum_wm

    @pl.when(i == n - 1)
    def _fin():
        nll = -(acc_ref[0] - acc_ref[1]) / float(_S)
        mse = acc_ref[2] / acc_ref[3]
        out_ref[0] = (1.0 - _LAMBDA_MSE) * nll + _LAMBDA_MSE * mse
        out_ref[1] = nll
        out_ref[2] = mse


def _tc_reduce(t, scores, lens, a, b_s):
    bs = 512
    grid = _S // bs
    out = pl.pallas_call(
        _tc_body,
        grid=(grid,),
        in_specs=[
            pl.BlockSpec((1, 1), lambda i: (0, 0), memory_space=pltpu.SMEM),
            pl.BlockSpec((bs, 1), lambda i: (i, 0)),
            pl.BlockSpec((bs, 1), lambda i: (i, 0)),
            pl.BlockSpec((bs, _K), lambda i: (i, 0)),
            pl.BlockSpec((bs, _K), lambda i: (i, 0)),
        ],
        out_specs=pl.BlockSpec(memory_space=pltpu.SMEM),
        out_shape=jax.ShapeDtypeStruct((3,), jnp.float32),
        scratch_shapes=[pltpu.SMEM((4,), jnp.float32)],
    )(
        jnp.asarray(a, jnp.float32).reshape(1, 1),
        lens.reshape(_S, 1),
        b_s.reshape(_S, 1),
        t,
        scores,
    )
    return out


def kernel(slates, scores, lens, theta, a, b_s):
    idx2d = slates.reshape(_ROWS, _IDXW)
    gathered = _sc_gather(idx2d, theta)
    t = gathered.reshape(_S, _K)
    out = _tc_reduce(t, scores, lens, a, b_s)
    return (out[0], out[1], out[2])
